# baseline (speedup 1.0000x reference)
import functools

import jax
import jax.numpy as jnp
from jax import lax
from jax.experimental import pallas as pl
from jax.experimental.pallas import tpu as pltpu
from jax.experimental.pallas import tpu_sc as plsc

_CUTOFF = 5.0
_NORM = (2.0 / _CUTOFF) ** 0.5
_NC = 2
_NS = 16
_NW = _NC * _NS
_CHUNK = 4000
_EPR = 64


def _sc_d2(ptab_flat, ei_flat, N):
    E = ei_flat.shape[0] // 2
    ew = E // _NW
    nchunks = ew // _CHUNK
    mesh = plsc.VectorSubcoreMesh(
        core_axis_name="c", subcore_axis_name="s",
        num_cores=_NC, num_subcores=_NS)

    @functools.partial(
        pl.kernel,
        out_type=jax.ShapeDtypeStruct((E,), jnp.float32),
        mesh=mesh,
        compiler_params=pltpu.CompilerParams(needs_layout_passes=False),
        scratch_types=[
            pltpu.VMEM((N,), jnp.float32),
            pltpu.VMEM((_CHUNK,), jnp.int32),
            pltpu.VMEM((_CHUNK,), jnp.int32),
            pltpu.VMEM((_CHUNK,), jnp.float32),
        ],
    )
    def sc_kernel(ptab_hbm, ei_hbm, d2_hbm, tab, sidx, didx, d2v):
        wid = lax.axis_index("s") * _NC + lax.axis_index("c")
        wbase = wid * ew

        for comp in range(3):
            pltpu.sync_copy(ptab_hbm.at[pl.ds(comp * N, N)], tab)

            def chunk_body(ci, carry):
                base = wbase + ci * _CHUNK
                pltpu.sync_copy(ei_hbm.at[pl.ds(base, _CHUNK)], sidx)
                pltpu.sync_copy(ei_hbm.at[pl.ds(E + base, _CHUNK)], didx)
                if comp > 0:
                    pltpu.sync_copy(d2_hbm.at[pl.ds(base, _CHUNK)], d2v)

                def vec_body(i, c2):
                    sl = pl.ds(i * 16, 16)
                    a = plsc.load_gather(tab, [sidx[sl]])
                    b = plsc.load_gather(tab, [didx[sl]])
                    d = a - b
                    if comp == 0:
                        d2v[sl] = d * d
                    else:
                        d2v[sl] = d2v[sl] + d * d
                    return c2

                lax.fori_loop(0, _CHUNK // 16, vec_body, 0)
                pltpu.sync_copy(d2v, d2_hbm.at[pl.ds(base, _CHUNK)])
                return carry

            lax.fori_loop(0, nchunks, chunk_body, 0)

    return sc_kernel(ptab_flat, ei_flat)


def _tc_rbf(d2, freqs):
    E = d2.shape[0]
    nb = freqs.shape[0]
    epr = _EPR
    L = epr * nb
    R = E // epr
    BR = 40
    grid = R // BR

    cols = jnp.arange(L, dtype=jnp.int32)
    rep = (cols[None, :] // nb == jnp.arange(epr, dtype=jnp.int32)[:, None])
    rep = rep.astype(jnp.float32)
    frep = rep * freqs[jnp.mod(cols, nb)][None, :]

    def body(d2_ref, frep_ref, rep_ref, o_ref):
        d2b = d2_ref[...]
        theta = jnp.sqrt(d2b) * (1.0 / _CUTOFF)
        invn = _NORM * jax.lax.rsqrt(d2b)
        arg = jnp.dot(theta, frep_ref[...],
                      precision=jax.lax.Precision.HIGHEST,
                      preferred_element_type=jnp.float32)
        invrep = jnp.dot(invn, rep_ref[...],
                         precision=jax.lax.Precision.HIGHEST,
                         preferred_element_type=jnp.float32)
        o_ref[...] = invrep * jnp.sin(arg)

    out = pl.pallas_call(
        body,
        grid=(grid,),
        in_specs=[pl.BlockSpec((BR, epr), lambda i: (i, 0)),
                  pl.BlockSpec((epr, L), lambda i: (0, 0)),
                  pl.BlockSpec((epr, L), lambda i: (0, 0))],
        out_specs=pl.BlockSpec((BR, L), lambda i: (i, 0)),
        out_shape=jax.ShapeDtypeStruct((R, L), jnp.float32),
    )(d2.reshape(R, epr), frep, rep)
    one = (freqs[0] * 0.0 + 1.0).astype(jnp.float32)
    return out.reshape(E, nb) * one


def kernel(pos, edge_index, freqs):
    N = pos.shape[0]
    ptab_flat = pos.T.reshape(-1)
    d2 = _sc_d2(ptab_flat, edge_index.reshape(-1), N)
    return _tc_rbf(d2, freqs)


# speedup vs baseline: 1.5583x; 1.5583x over previous
import functools

import jax
import jax.numpy as jnp
from jax import lax
from jax.experimental import pallas as pl
from jax.experimental.pallas import tpu as pltpu
from jax.experimental.pallas import tpu_sc as plsc

_CUTOFF = 5.0
_NORM = (2.0 / _CUTOFF) ** 0.5
_NC = 2
_NS = 16
_NW = _NC * _NS
_CHUNK = 4000
_EPR = 64


def _sc_d2(ptab_flat, ei_flat, N):
    E = ei_flat.shape[0] // 2
    ew = E // _NW
    nchunks = ew // _CHUNK
    mesh = plsc.VectorSubcoreMesh(
        core_axis_name="c", subcore_axis_name="s",
        num_cores=_NC, num_subcores=_NS)

    @functools.partial(
        pl.kernel,
        out_type=jax.ShapeDtypeStruct((E,), jnp.float32),
        mesh=mesh,
        compiler_params=pltpu.CompilerParams(needs_layout_passes=False),
        scratch_types=[
            pltpu.VMEM((N,), jnp.float32),
            pltpu.VMEM((_CHUNK,), jnp.int32),
            pltpu.VMEM((_CHUNK,), jnp.int32),
            pltpu.VMEM((_CHUNK,), jnp.float32),
        ],
    )
    def sc_kernel(ptab_hbm, ei_hbm, d2_hbm, tab, sidx, didx, d2v):
        wid = lax.axis_index("s") * _NC + lax.axis_index("c")
        wbase = wid * ew

        for comp in range(3):
            pltpu.sync_copy(ptab_hbm.at[pl.ds(comp * N, N)], tab)

            def chunk_body(ci, carry):
                base = wbase + ci * _CHUNK
                pltpu.sync_copy(ei_hbm.at[pl.ds(base, _CHUNK)], sidx)
                pltpu.sync_copy(ei_hbm.at[pl.ds(E + base, _CHUNK)], didx)
                if comp > 0:
                    pltpu.sync_copy(d2_hbm.at[pl.ds(base, _CHUNK)], d2v)

                def vec_body(i, c2):
                    sl = pl.ds(i * 16, 16)
                    a = plsc.load_gather(tab, [sidx[sl]])
                    b = plsc.load_gather(tab, [didx[sl]])
                    d = a - b
                    if comp == 0:
                        d2v[sl] = d * d
                    else:
                        d2v[sl] = d2v[sl] + d * d
                    return c2

                lax.fori_loop(0, _CHUNK // 16, vec_body, 0)
                pltpu.sync_copy(d2v, d2_hbm.at[pl.ds(base, _CHUNK)])
                return carry

            lax.fori_loop(0, nchunks, chunk_body, 0)

    return sc_kernel(ptab_flat, ei_flat)


def _tc_rbf(d2, freqs):
    E = d2.shape[0]
    nb = freqs.shape[0]
    SUB, LN = 8, 128
    BE = SUB * LN
    grid = E // BE

    def body(d2_ref, f_ref, o_ref):
        d2b = d2_ref[...]
        f1 = f_ref[0, 0]
        inv = _NORM * jax.lax.rsqrt(d2b)
        phi = jnp.sqrt(d2b) * (f1 * (1.0 / _CUTOFF))
        s1 = jnp.sin(phi) * inv
        c2 = 2.0 * jnp.cos(phi)
        sk = [s1]
        prev2, prev1 = jnp.zeros_like(s1), s1
        for _ in range(1, nb):
            cur = c2 * prev1 - prev2
            sk.append(cur)
            prev2, prev1 = prev1, cur
        for r in range(SUB):
            m = jnp.concatenate([s[r:r + 1, :] for s in sk], axis=0)
            o_ref[pl.ds(r * LN, LN), :] = m.T

    return pl.pallas_call(
        body,
        grid=(grid,),
        in_specs=[pl.BlockSpec((SUB, LN), lambda i: (i, 0)),
                  pl.BlockSpec((1, nb), lambda i: (0, 0))],
        out_specs=pl.BlockSpec((BE, nb), lambda i: (i, 0)),
        out_shape=jax.ShapeDtypeStruct((E, nb), jnp.float32),
    )(d2.reshape(E // LN, LN), freqs.reshape(1, nb))


def kernel(pos, edge_index, freqs):
    N = pos.shape[0]
    ptab_flat = pos.T.reshape(-1)
    d2 = _sc_d2(ptab_flat, edge_index.reshape(-1), N)
    return _tc_rbf(d2, freqs)


# speedup vs baseline: 2.5084x; 1.6097x over previous
import functools

import jax
import jax.numpy as jnp
from jax import lax
from jax.experimental import pallas as pl
from jax.experimental.pallas import tpu as pltpu
from jax.experimental.pallas import tpu_sc as plsc

_CUTOFF = 5.0
_NORM = (2.0 / _CUTOFF) ** 0.5
_NC = 2
_NS = 16
_NW = _NC * _NS
_CHUNK = 4000
_EPR = 64


def _sc_d2(ptab_flat, ei_flat, N):
    E = ei_flat.shape[0] // 2
    ew = E // _NW
    nchunks = ew // _CHUNK
    mesh = plsc.VectorSubcoreMesh(
        core_axis_name="c", subcore_axis_name="s",
        num_cores=_NC, num_subcores=_NS)

    @functools.partial(
        pl.kernel,
        out_type=jax.ShapeDtypeStruct((E,), jnp.float32),
        mesh=mesh,
        compiler_params=pltpu.CompilerParams(needs_layout_passes=False),
        scratch_types=[
            pltpu.VMEM((N,), jnp.float32),
            pltpu.VMEM((_CHUNK,), jnp.int32),
            pltpu.VMEM((_CHUNK,), jnp.int32),
            pltpu.VMEM((_CHUNK,), jnp.float32),
        ],
    )
    def sc_kernel(ptab_hbm, ei_hbm, d2_hbm, tab, sidx, didx, d2v):
        wid = lax.axis_index("s") * _NC + lax.axis_index("c")
        wbase = wid * ew

        for comp in range(3):
            pltpu.sync_copy(ptab_hbm.at[pl.ds(comp * N, N)], tab)

            def chunk_body(ci, carry):
                base = wbase + ci * _CHUNK
                pltpu.sync_copy(ei_hbm.at[pl.ds(base, _CHUNK)], sidx)
                pltpu.sync_copy(ei_hbm.at[pl.ds(E + base, _CHUNK)], didx)
                if comp > 0:
                    pltpu.sync_copy(d2_hbm.at[pl.ds(base, _CHUNK)], d2v)

                def vec_body(i, c2):
                    sl = pl.ds(i * 16, 16)
                    a = plsc.load_gather(tab, [sidx[sl]])
                    b = plsc.load_gather(tab, [didx[sl]])
                    d = a - b
                    if comp == 0:
                        d2v[sl] = d * d
                    else:
                        d2v[sl] = d2v[sl] + d * d
                    return c2

                lax.fori_loop(0, _CHUNK // 16, vec_body, 0)
                pltpu.sync_copy(d2v, d2_hbm.at[pl.ds(base, _CHUNK)])
                return carry

            lax.fori_loop(0, nchunks, chunk_body, 0)

    return sc_kernel(ptab_flat, ei_flat)


def _tc_rbf(d2, freqs):
    E = d2.shape[0]
    nb = freqs.shape[0]
    SUB, LN = 40, 128
    BE = SUB * LN
    grid = E // BE

    def body(d2_ref, f_ref, o_ref):
        f1 = f_ref[0, 0]
        for c in range(SUB // 8):
            d2b = d2_ref[pl.ds(c * 8, 8), :]
            inv = _NORM * jax.lax.rsqrt(d2b)
            phi = jnp.sqrt(d2b) * (f1 * (1.0 / _CUTOFF))
            s1 = jnp.sin(phi) * inv
            c2 = 2.0 * jnp.cos(phi)
            sk = [s1]
            prev2, prev1 = jnp.zeros_like(s1), s1
            for _ in range(1, nb):
                cur = c2 * prev1 - prev2
                sk.append(cur)
                prev2, prev1 = prev1, cur
            for r in range(8):
                m = jnp.concatenate([s[r:r + 1, :] for s in sk], axis=0)
                o_ref[pl.ds((c * 8 + r) * LN, LN), :] = m.T

    return pl.pallas_call(
        body,
        grid=(grid,),
        in_specs=[pl.BlockSpec((SUB, LN), lambda i: (i, 0)),
                  pl.BlockSpec((1, nb), lambda i: (0, 0))],
        out_specs=pl.BlockSpec((BE, nb), lambda i: (i, 0)),
        out_shape=jax.ShapeDtypeStruct((E, nb), jnp.float32),
    )(d2.reshape(E // LN, LN), freqs.reshape(1, nb))


def kernel(pos, edge_index, freqs):
    N = pos.shape[0]
    ptab_flat = pos.T.reshape(-1)
    d2 = _sc_d2(ptab_flat, edge_index.reshape(-1), N)
    return _tc_rbf(d2, freqs)


# speedup vs baseline: 2.8331x; 1.1294x over previous
import functools

import jax
import jax.numpy as jnp
from jax import lax
from jax.experimental import pallas as pl
from jax.experimental.pallas import tpu as pltpu
from jax.experimental.pallas import tpu_sc as plsc

_CUTOFF = 5.0
_NORM = (2.0 / _CUTOFF) ** 0.5
_NC = 2
_NS = 16
_NW = _NC * _NS
_CHUNK = 4000
_EPR = 64


def _sc_d2(ptab_flat, ei_flat, N):
    E = ei_flat.shape[0] // 2
    ew = E // _NW
    nchunks = ew // _CHUNK
    mesh = plsc.VectorSubcoreMesh(
        core_axis_name="c", subcore_axis_name="s",
        num_cores=_NC, num_subcores=_NS)

    @functools.partial(
        pl.kernel,
        out_type=jax.ShapeDtypeStruct((E,), jnp.float32),
        mesh=mesh,
        compiler_params=pltpu.CompilerParams(needs_layout_passes=False),
        scratch_types=[
            pltpu.VMEM((N,), jnp.float32),
            pltpu.VMEM((_CHUNK,), jnp.int32),
            pltpu.VMEM((_CHUNK,), jnp.int32),
            pltpu.VMEM((_CHUNK,), jnp.float32),
        ],
    )
    def sc_kernel(ptab_hbm, ei_hbm, d2_hbm, tab, sidx, didx, d2v):
        wid = lax.axis_index("s") * _NC + lax.axis_index("c")
        wbase = wid * ew

        for comp in range(3):
            pltpu.sync_copy(ptab_hbm.at[pl.ds(comp * N, N)], tab)

            def chunk_body(ci, carry):
                base = wbase + ci * _CHUNK
                pltpu.sync_copy(ei_hbm.at[pl.ds(base, _CHUNK)], sidx)
                pltpu.sync_copy(ei_hbm.at[pl.ds(E + base, _CHUNK)], didx)
                if comp > 0:
                    pltpu.sync_copy(d2_hbm.at[pl.ds(base, _CHUNK)], d2v)

                def vec_body(i, c2):
                    sl = pl.ds(i * 16, 16)
                    a = plsc.load_gather(tab, [sidx[sl]])
                    b = plsc.load_gather(tab, [didx[sl]])
                    d = a - b
                    if comp == 0:
                        d2v[sl] = d * d
                    else:
                        d2v[sl] = d2v[sl] + d * d
                    return c2

                lax.fori_loop(0, _CHUNK // 16, vec_body, 0)
                pltpu.sync_copy(d2v, d2_hbm.at[pl.ds(base, _CHUNK)])
                return carry

            lax.fori_loop(0, nchunks, chunk_body, 0)

    return sc_kernel(ptab_flat, ei_flat)


def _tc_rbf(d2, freqs):
    E = d2.shape[0]
    nb = freqs.shape[0]
    SUB, LN = 200, 128
    BE = SUB * LN
    grid = E // BE

    def body(d2_ref, f_ref, o_ref):
        f1 = f_ref[0, 0]
        for c in range(SUB // 8):
            d2b = d2_ref[pl.ds(c * 8, 8), :]
            inv = _NORM * jax.lax.rsqrt(d2b)
            phi = jnp.sqrt(d2b) * (f1 * (1.0 / _CUTOFF))
            s1 = jnp.sin(phi) * inv
            c2 = 2.0 * jnp.cos(phi)
            sk = [s1]
            prev2, prev1 = jnp.zeros_like(s1), s1
            for _ in range(1, nb):
                cur = c2 * prev1 - prev2
                sk.append(cur)
                prev2, prev1 = prev1, cur
            for r in range(8):
                m = jnp.concatenate([s[r:r + 1, :] for s in sk], axis=0)
                o_ref[pl.ds((c * 8 + r) * LN, LN), :] = m.T

    return pl.pallas_call(
        body,
        grid=(grid,),
        in_specs=[pl.BlockSpec((SUB, LN), lambda i: (i, 0)),
                  pl.BlockSpec((1, nb), lambda i: (0, 0))],
        out_specs=pl.BlockSpec((BE, nb), lambda i: (i, 0)),
        out_shape=jax.ShapeDtypeStruct((E, nb), jnp.float32),
    )(d2.reshape(E // LN, LN), freqs.reshape(1, nb))


def kernel(pos, edge_index, freqs):
    N = pos.shape[0]
    ptab_flat = pos.T.reshape(-1)
    d2 = _sc_d2(ptab_flat, edge_index.reshape(-1), N)
    return _tc_rbf(d2, freqs)


# speedup vs baseline: 3.1376x; 1.1075x over previous
import functools

import jax
import jax.numpy as jnp
from jax import lax
from jax.experimental import pallas as pl
from jax.experimental.pallas import tpu as pltpu
from jax.experimental.pallas import tpu_sc as plsc

_CUTOFF = 5.0
_NORM = (2.0 / _CUTOFF) ** 0.5
_NC = 2
_NS = 16
_NW = _NC * _NS
_CHUNK = 4000
_EPR = 64


def _sc_d2(ptab_flat, ei_flat, N):
    E = ei_flat.shape[0] // 2
    ew = E // _NW
    nchunks = ew // _CHUNK
    npairs = nchunks // 2
    mesh = plsc.VectorSubcoreMesh(
        core_axis_name="c", subcore_axis_name="s",
        num_cores=_NC, num_subcores=_NS)

    @functools.partial(
        pl.kernel,
        out_type=jax.ShapeDtypeStruct((3 * E,), jnp.float32),
        mesh=mesh,
        compiler_params=pltpu.CompilerParams(needs_layout_passes=False),
        scratch_types=[
            pltpu.VMEM((N,), jnp.float32),
            pltpu.VMEM((_CHUNK,), jnp.int32),
            pltpu.VMEM((_CHUNK,), jnp.int32),
            pltpu.VMEM((_CHUNK,), jnp.int32),
            pltpu.VMEM((_CHUNK,), jnp.int32),
            pltpu.VMEM((_CHUNK,), jnp.float32),
            pltpu.VMEM((_CHUNK,), jnp.float32),
            pltpu.SemaphoreType.DMA,
            pltpu.SemaphoreType.DMA,
            pltpu.SemaphoreType.DMA,
            pltpu.SemaphoreType.DMA,
            pltpu.SemaphoreType.DMA,
            pltpu.SemaphoreType.DMA,
        ],
    )
    def sc_kernel(ptab_hbm, ei_hbm, d2_hbm, tab, s0, s1, d0, d1, o0, o1,
                  ss0, ss1, sd0, sd1, so0, so1):
        wid = lax.axis_index("s") * _NC + lax.axis_index("c")
        wbase = wid * ew
        sbufs, dbufs, obufs = (s0, s1), (d0, d1), (o0, o1)
        ssems, dsems, osems = (ss0, ss1), (sd0, sd1), (so0, so1)

        def fire(ci, b):
            base = wbase + ci * _CHUNK
            pltpu.async_copy(ei_hbm.at[pl.ds(base, _CHUNK)], sbufs[b], ssems[b])
            pltpu.async_copy(ei_hbm.at[pl.ds(E + base, _CHUNK)],
                             dbufs[b], dsems[b])

        def compute(comp, ci, b, drain_out):
            if drain_out:
                pltpu.make_async_copy(
                    obufs[b], d2_hbm.at[pl.ds(wbase, _CHUNK)], osems[b]).wait()
            pltpu.make_async_copy(
                ei_hbm.at[pl.ds(wbase, _CHUNK)], sbufs[b], ssems[b]).wait()
            pltpu.make_async_copy(
                ei_hbm.at[pl.ds(wbase, _CHUNK)], dbufs[b], dsems[b]).wait()
            sb, db, ob = sbufs[b], dbufs[b], obufs[b]

            def vec_body(i, c2):
                sl = pl.ds(i * 16, 16)
                a = plsc.load_gather(tab, [sb[sl]])
                bv = plsc.load_gather(tab, [db[sl]])
                d = a - bv
                ob[sl] = d * d
                return c2

            lax.fori_loop(0, _CHUNK // 16, vec_body, 0)
            pltpu.async_copy(
                ob, d2_hbm.at[pl.ds(comp * E + wbase + ci * _CHUNK, _CHUNK)],
                osems[b])

        for comp in range(3):
            pltpu.sync_copy(ptab_hbm.at[pl.ds(comp * N, N)], tab)
            fire(0, 0)
            fire(1, 1)
            if comp == 0:
                compute(comp, 0, 0, drain_out=False)
                fire(2, 0)
                compute(comp, 1, 1, drain_out=False)
                fire(3, 1)
                t0 = 1
            else:
                t0 = 0

            def pair_body(t, carry):
                ci0 = 2 * t
                compute(comp, ci0, 0, drain_out=True)
                fire(ci0 + 2, 0)
                compute(comp, ci0 + 1, 1, drain_out=True)

                @pl.when(t < npairs - 1)
                def _():
                    fire(ci0 + 3, 1)
                return carry

            lax.fori_loop(t0, npairs, pair_body, 0)
            compute(comp, nchunks - 1, 0, drain_out=True)

        pltpu.make_async_copy(
            o0, d2_hbm.at[pl.ds(wbase, _CHUNK)], so0).wait()
        pltpu.make_async_copy(
            o1, d2_hbm.at[pl.ds(wbase, _CHUNK)], so1).wait()

    return sc_kernel(ptab_flat, ei_flat)


def _tc_rbf(d2, freqs):
    E = d2.shape[0] // 3
    nb = freqs.shape[0]
    SUB, LN = 200, 128
    BE = SUB * LN
    grid = E // BE
    d2v = d2.reshape(3, E // LN, LN)

    def body(dx_ref, dy_ref, dz_ref, f_ref, o_ref):
        f1 = f_ref[0, 0]
        for c in range(SUB // 8):
            csl = pl.ds(c * 8, 8)
            d2b = (dx_ref[0, csl, :] + dy_ref[0, csl, :]
                   + dz_ref[0, csl, :])
            inv = _NORM * jax.lax.rsqrt(d2b)
            phi = jnp.sqrt(d2b) * (f1 * (1.0 / _CUTOFF))
            s1 = jnp.sin(phi) * inv
            c2 = 2.0 * jnp.cos(phi)
            sk = [s1]
            prev2, prev1 = jnp.zeros_like(s1), s1
            for _ in range(1, nb):
                cur = c2 * prev1 - prev2
                sk.append(cur)
                prev2, prev1 = prev1, cur
            for r in range(8):
                m = jnp.concatenate([s[r:r + 1, :] for s in sk], axis=0)
                o_ref[pl.ds((c * 8 + r) * LN, LN), :] = m.T

    return pl.pallas_call(
        body,
        grid=(grid,),
        in_specs=[pl.BlockSpec((1, SUB, LN), lambda i: (0, i, 0)),
                  pl.BlockSpec((1, SUB, LN), lambda i: (1, i, 0)),
                  pl.BlockSpec((1, SUB, LN), lambda i: (2, i, 0)),
                  pl.BlockSpec((1, nb), lambda i: (0, 0))],
        out_specs=pl.BlockSpec((BE, nb), lambda i: (i, 0)),
        out_shape=jax.ShapeDtypeStruct((E, nb), jnp.float32),
    )(d2v, d2v, d2v, freqs.reshape(1, nb))


def kernel(pos, edge_index, freqs):
    N = pos.shape[0]
    ptab_flat = pos.T.reshape(-1)
    d2 = _sc_d2(ptab_flat, edge_index.reshape(-1), N)
    return _tc_rbf(d2, freqs)
